# baseline (speedup 1.0000x reference)
import functools

import jax
import jax.numpy as jnp
from jax import lax
from jax.experimental import pallas as pl
from jax.experimental.pallas import tpu as pltpu
from jax.experimental.pallas import tpu_sc as plsc

N = 10000
E = 320000
D = 128

NC = 2
NS = 16
NW = NC * NS
EPW = E // NW
CH = 125
NCH = EPW // CH
NP = 10240
RPS = NP // NS
DW = 16

_sc_mesh = functools.partial(
    plsc.VectorSubcoreMesh, core_axis_name="c", subcore_axis_name="s"
)


def _deg_body(g4_hbm, zeros_hbm, out_hbm, acc, ones_v, idx2):
    cid = lax.axis_index("c")
    sid = lax.axis_index("s")
    wid = cid * NS + sid

    pltpu.sync_copy(zeros_hbm, acc.at[pl.ds(sid * RPS, RPS)])

    ones16 = jnp.ones((16,), jnp.float32)
    for i in range(128 // 16):
        ones_v[pl.ds(i * 16, 16)] = ones16

    pltpu.sync_copy(g4_hbm.at[1, wid], idx2)

    plsc.subcore_barrier()

    def chunk(c, carry):
        pltpu.sync_copy(ones_v.at[pl.ds(0, CH)], acc.at[idx2.at[c]], add=True)
        return carry

    lax.fori_loop(0, NCH, chunk, 0)

    plsc.subcore_barrier()
    pltpu.sync_copy(
        acc.at[pl.ds(sid * RPS, RPS)],
        out_hbm.at[pl.ds(cid * NP + sid * RPS, RPS)],
    )


_deg_call = pl.kernel(
    _deg_body,
    out_type=jax.ShapeDtypeStruct((NC * NP,), jnp.float32),
    mesh=_sc_mesh(),
    scratch_types=[
        pltpu.VMEM_SHARED((NP,), jnp.float32),
        pltpu.VMEM((128,), jnp.float32),
        pltpu.VMEM((NCH, CH), jnp.int32),
    ],
)


def _agg_body(gi_hbm, hs_hbm, zeros_hbm, out_hbm, acc, ibuf, rows,
              gsem, ssem, isem):
    cid = lax.axis_index("c")
    sid = lax.axis_index("s")
    wid = cid * NS + sid

    @pl.when(cid == 0)
    def _():
        pltpu.sync_copy(
            hs_hbm.at[pl.ds(sid * RPS, RPS)], acc.at[pl.ds(sid * RPS, RPS)]
        )

    @pl.when(cid == 1)
    def _():
        pltpu.sync_copy(zeros_hbm, acc.at[pl.ds(sid * RPS, RPS)])

    def issue_idx(cc):
        pltpu.async_copy(gi_hbm.at[wid, cc], ibuf.at[lax.rem(cc, 4)], isem)

    def wait_idx(cc):
        pltpu.make_async_copy(
            gi_hbm.at[wid, 0], ibuf.at[lax.rem(cc, 4)], isem
        ).wait()

    def issue_gather(cc, r):
        pltpu.async_copy(hs_hbm.at[ibuf.at[lax.rem(cc, 4), 0]], rows.at[r], gsem)

    def wait_gather(cc, r):
        pltpu.make_async_copy(
            hs_hbm.at[ibuf.at[lax.rem(cc, 4), 0]], rows.at[r], gsem
        ).wait()

    def issue_scatter(cc, r):
        pltpu.async_copy(
            rows.at[r], acc.at[ibuf.at[lax.rem(cc, 4), 1]], ssem, add=True
        )

    def wait_scatter(cc, r):
        pltpu.make_async_copy(
            rows.at[r], acc.at[ibuf.at[lax.rem(cc, 4), 1]], ssem
        ).wait()

    plsc.subcore_barrier()

    issue_idx(0)
    issue_idx(1)
    wait_idx(0)
    issue_gather(0, 0)

    def pair(c, carry):
        for b in range(2):
            cc = 2 * c + b
            wait_gather(cc, b)

            @pl.when(cc >= 1)
            def _():
                wait_scatter(cc - 1, 1 - b)

            @pl.when(cc + 1 < NCH)
            def _():
                wait_idx(cc + 1)
                issue_gather(cc + 1, 1 - b)

            issue_scatter(cc, b)

            @pl.when(cc + 2 < NCH)
            def _():
                issue_idx(cc + 2)
        return carry

    lax.fori_loop(0, NCH // 2, pair, 0)

    wait_scatter(NCH - 1, 1)

    plsc.subcore_barrier()
    pltpu.sync_copy(
        acc.at[pl.ds(sid * RPS, RPS)],
        out_hbm.at[cid, pl.ds(sid * RPS, RPS)],
    )


_agg_call = pl.kernel(
    _agg_body,
    out_type=jax.ShapeDtypeStruct((NC, NP, D), jnp.float32),
    mesh=_sc_mesh(),
    scratch_types=[
        pltpu.VMEM_SHARED((NP, D), jnp.float32),
        pltpu.VMEM((4, 2, CH), jnp.int32),
        pltpu.VMEM((2, CH, D), jnp.float32),
        pltpu.SemaphoreType.DMA,
        pltpu.SemaphoreType.DMA,
        pltpu.SemaphoreType.DMA,
    ],
)


RB = 2000


def _dinv_of(deg_blk):
    return lax.rsqrt(deg_blk[:, 0:1] + deg_blk[:, 1:2] + 1.0)


def _mm_body(deg_ref, x_ref, w_ref, hs_ref):
    dinv = _dinv_of(deg_ref[...])
    h = jnp.dot(x_ref[...], w_ref[...], preferred_element_type=jnp.float32)
    hs_ref[...] = h * dinv


RB2 = 2048


def _mm_call(deg2, x, w):
    return pl.pallas_call(
        _mm_body,
        grid=(NP // RB2,),
        in_specs=[
            pl.BlockSpec((RB2, 2), lambda i: (i, 0)),
            pl.BlockSpec((RB2, D), lambda i: (i, 0)),
            pl.BlockSpec((D, D), lambda i: (0, 0)),
        ],
        out_specs=pl.BlockSpec((RB2, D), lambda i: (i, 0)),
        out_shape=jax.ShapeDtypeStruct((NP, D), jnp.float32),
    )(deg2, x, w)


def _fc_body(deg_ref, p0_ref, p1_ref, bc_ref, w1_ref, b1_ref,
             w2_ref, b2_ref, out_ref):
    dinv = _dinv_of(deg_ref[...])
    t = (p0_ref[0] + p1_ref[0]) * dinv + bc_ref[...]
    t = jnp.maximum(t, 0.0)
    t = jnp.dot(t, w1_ref[...], preferred_element_type=jnp.float32) + b1_ref[...]
    t = jnp.maximum(t, 0.0)
    out_ref[...] = (
        jnp.dot(t, w2_ref[...], preferred_element_type=jnp.float32) + b2_ref[...]
    )


def _fc_call(deg2, aggp, bc, w1, b1, w2, b2):
    row = lambda i: (i, 0)
    full = lambda i: (0, 0)
    return pl.pallas_call(
        _fc_body,
        grid=(N // RB,),
        in_specs=[
            pl.BlockSpec((RB, 2), row),
            pl.BlockSpec((1, RB, D), lambda i: (0, i, 0)),
            pl.BlockSpec((1, RB, D), lambda i: (1, i, 0)),
            pl.BlockSpec((1, D), full),
            pl.BlockSpec((D, D), full),
            pl.BlockSpec((1, D), full),
            pl.BlockSpec((D, D), full),
            pl.BlockSpec((1, D), full),
        ],
        out_specs=pl.BlockSpec((RB, D), row),
        out_shape=jax.ShapeDtypeStruct((N, D), jnp.float32),
    )(deg2, aggp, aggp, bc, w1, b1, w2, b2)


def kernel(x, g, W_conv, b_conv, W_fc, b_fc, W_fc2, b_fc2):
    g3 = g.reshape(2, NW, NCH, CH)
    gi = jnp.stack((g3[0], g3[1]), axis=2)
    zeros_deg = jnp.zeros((RPS,), jnp.float32)
    zeros_agg = jnp.zeros((RPS, D), jnp.float32)

    degp = _deg_call(g3, zeros_deg)
    deg2 = degp.reshape(NC, NP)[:, :N].T
    hs = _mm_call(deg2, x, W_conv)
    aggp = _agg_call(gi, hs, zeros_agg)
    return _fc_call(
        deg2, aggp,
        b_conv[None, :], W_fc, b_fc[None, :], W_fc2, b_fc2[None, :],
    )


# speedup vs baseline: 1.0718x; 1.0718x over previous
import functools

import jax
import jax.numpy as jnp
from jax import lax
from jax.experimental import pallas as pl
from jax.experimental.pallas import tpu as pltpu
from jax.experimental.pallas import tpu_sc as plsc

N = 10000
E = 320000
D = 128

NC = 2
NS = 16
NW = NC * NS
EPW = E // NW
CH = 125
NCH = EPW // CH
NP = 10240
RPS = NP // NS
DW = 16

_sc_mesh = functools.partial(
    plsc.VectorSubcoreMesh, core_axis_name="c", subcore_axis_name="s"
)


def _deg_body(g4_hbm, zeros_hbm, out_hbm, acc, ones_v, idx2):
    cid = lax.axis_index("c")
    sid = lax.axis_index("s")
    wid = cid * NS + sid

    pltpu.sync_copy(zeros_hbm, acc.at[pl.ds(sid * RPS, RPS)])

    ones16 = jnp.ones((16,), jnp.float32)
    for i in range(128 // 16):
        ones_v[pl.ds(i * 16, 16)] = ones16

    pltpu.sync_copy(g4_hbm.at[1, wid], idx2)

    plsc.subcore_barrier()

    def chunk(c, carry):
        pltpu.sync_copy(ones_v.at[pl.ds(0, CH)], acc.at[idx2.at[c]], add=True)
        return carry

    lax.fori_loop(0, NCH, chunk, 0)

    plsc.subcore_barrier()
    pltpu.sync_copy(
        acc.at[pl.ds(sid * RPS, RPS)],
        out_hbm.at[pl.ds(cid * NP + sid * RPS, RPS)],
    )


_deg_call = pl.kernel(
    _deg_body,
    out_type=jax.ShapeDtypeStruct((NC * NP,), jnp.float32),
    mesh=_sc_mesh(),
    scratch_types=[
        pltpu.VMEM_SHARED((NP,), jnp.float32),
        pltpu.VMEM((128,), jnp.float32),
        pltpu.VMEM((NCH, CH), jnp.int32),
    ],
)


def _agg_body(g4_hbm, hs_hbm, zeros_hbm, out_hbm,
              acc, ibuf, rows, gsem, isem):
    cid = lax.axis_index("c")
    sid = lax.axis_index("s")
    wid = cid * NS + sid

    @pl.when(cid == 0)
    def _():
        pltpu.sync_copy(
            hs_hbm.at[pl.ds(sid * RPS, RPS)], acc.at[pl.ds(sid * RPS, RPS)]
        )

    @pl.when(cid == 1)
    def _():
        pltpu.sync_copy(zeros_hbm, acc.at[pl.ds(sid * RPS, RPS)])

    def issue_idx(cc, b):
        pltpu.async_copy(g4_hbm.at[0, wid, cc], ibuf.at[b, 0], isem)
        pltpu.async_copy(g4_hbm.at[1, wid, cc], ibuf.at[b, 1], isem)

    def wait_idx(b):
        pltpu.make_async_copy(g4_hbm.at[0, wid, 0], ibuf.at[b, 0], isem).wait()
        pltpu.make_async_copy(g4_hbm.at[0, wid, 0], ibuf.at[b, 1], isem).wait()

    plsc.subcore_barrier()

    issue_idx(0, 0)
    issue_idx(1, 1)
    wait_idx(0)
    pltpu.async_copy(hs_hbm.at[ibuf.at[0, 0]], rows.at[0], gsem)

    def pair(c, carry):
        for b in range(2):
            cc = 2 * c + b
            pltpu.make_async_copy(
                hs_hbm.at[ibuf.at[b, 0]], rows.at[b], gsem
            ).wait()

            @pl.when(cc + 1 < NCH)
            def _():
                wait_idx(1 - b)
                pltpu.async_copy(
                    hs_hbm.at[ibuf.at[1 - b, 0]], rows.at[1 - b], gsem
                )

            pltpu.sync_copy(rows.at[b], acc.at[ibuf.at[b, 1]], add=True)

            @pl.when(cc + 2 < NCH)
            def _():
                issue_idx(cc + 2, b)
        return carry

    lax.fori_loop(0, NCH // 2, pair, 0)

    plsc.subcore_barrier()
    pltpu.sync_copy(
        acc.at[pl.ds(sid * RPS, RPS)],
        out_hbm.at[cid, pl.ds(sid * RPS, RPS)],
    )


_agg_call = pl.kernel(
    _agg_body,
    out_type=jax.ShapeDtypeStruct((NC, NP, D), jnp.float32),
    mesh=_sc_mesh(),
    scratch_types=[
        pltpu.VMEM_SHARED((NP, D), jnp.float32),
        pltpu.VMEM((2, 2, CH), jnp.int32),
        pltpu.VMEM((2, CH, D), jnp.float32),
        pltpu.SemaphoreType.DMA,
        pltpu.SemaphoreType.DMA,
    ],
)


RB = 2000


def _dinv_of(deg_blk):
    return lax.rsqrt(deg_blk[:, 0:1] + deg_blk[:, 1:2] + 1.0)


def _mm_body(deg_ref, x_ref, w_ref, hs_ref):
    dinv = _dinv_of(deg_ref[...])
    h = jnp.dot(x_ref[...], w_ref[...], preferred_element_type=jnp.float32)
    hs_ref[...] = h * dinv


RB2 = 2048


def _mm_call(deg2, x, w):
    return pl.pallas_call(
        _mm_body,
        grid=(NP // RB2,),
        in_specs=[
            pl.BlockSpec((RB2, 2), lambda i: (i, 0)),
            pl.BlockSpec((RB2, D), lambda i: (i, 0)),
            pl.BlockSpec((D, D), lambda i: (0, 0)),
        ],
        out_specs=pl.BlockSpec((RB2, D), lambda i: (i, 0)),
        out_shape=jax.ShapeDtypeStruct((NP, D), jnp.float32),
    )(deg2, x, w)


def _fc_body(deg_ref, p0_ref, p1_ref, bc_ref, w1_ref, b1_ref,
             w2_ref, b2_ref, out_ref):
    dinv = _dinv_of(deg_ref[...])
    t = (p0_ref[0] + p1_ref[0]) * dinv + bc_ref[...]
    t = jnp.maximum(t, 0.0)
    t = jnp.dot(t, w1_ref[...], preferred_element_type=jnp.float32) + b1_ref[...]
    t = jnp.maximum(t, 0.0)
    out_ref[...] = (
        jnp.dot(t, w2_ref[...], preferred_element_type=jnp.float32) + b2_ref[...]
    )


def _fc_call(deg2, aggp, bc, w1, b1, w2, b2):
    row = lambda i: (i, 0)
    full = lambda i: (0, 0)
    return pl.pallas_call(
        _fc_body,
        grid=(N // RB,),
        in_specs=[
            pl.BlockSpec((RB, 2), row),
            pl.BlockSpec((1, RB, D), lambda i: (0, i, 0)),
            pl.BlockSpec((1, RB, D), lambda i: (1, i, 0)),
            pl.BlockSpec((1, D), full),
            pl.BlockSpec((D, D), full),
            pl.BlockSpec((1, D), full),
            pl.BlockSpec((D, D), full),
            pl.BlockSpec((1, D), full),
        ],
        out_specs=pl.BlockSpec((RB, D), row),
        out_shape=jax.ShapeDtypeStruct((N, D), jnp.float32),
    )(deg2, aggp, aggp, bc, w1, b1, w2, b2)


def kernel(x, g, W_conv, b_conv, W_fc, b_fc, W_fc2, b_fc2):
    g4 = g.reshape(2, NW, NCH, CH)
    zeros_deg = jnp.zeros((RPS,), jnp.float32)
    zeros_agg = jnp.zeros((RPS, D), jnp.float32)

    degp = _deg_call(g4, zeros_deg)
    deg2 = degp.reshape(NC, NP)[:, :N].T
    hs = _mm_call(deg2, x, W_conv)
    aggp = _agg_call(g4, hs, zeros_agg)
    return _fc_call(
        deg2, aggp,
        b_conv[None, :], W_fc, b_fc[None, :], W_fc2, b_fc2[None, :],
    )
